# baseline (speedup 1.0000x reference)
import functools

import jax
import jax.numpy as jnp
from jax import lax
from jax.experimental import pallas as pl
from jax.experimental.pallas import tpu as pltpu
from jax.experimental.pallas import tpu_sc as plsc

N = 10000
E = 160000
D = 256
DH = 128
NC = 2
NS = 16
ROWS_PER_TILE = N // NS
EDGES_PER_TILE = E // NS
K = 80
NCHUNK = EDGES_PER_TILE // K
CNT_EPT = E // (NC * NS)
ZR = 125

_mesh = plsc.VectorSubcoreMesh(core_axis_name="c", subcore_axis_name="s")


HCR = 80
MT = 5


def _make_segsum(with_count):
    out_type = [
        jax.ShapeDtypeStruct((N, DH), jnp.float32),
        jax.ShapeDtypeStruct((N, DH), jnp.float32),
    ]
    scratch = [
        pltpu.VMEM((K, DH), jnp.float32),
        pltpu.VMEM((K, DH), jnp.float32),
        pltpu.VMEM((K,), jnp.int32),
        pltpu.VMEM((K,), jnp.int32),
        pltpu.VMEM((K,), jnp.int32),
        pltpu.VMEM((K,), jnp.int32),
        pltpu.VMEM_SHARED((N, DH), jnp.float32),
        pltpu.SemaphoreType.DMA,
        pltpu.SemaphoreType.DMA,
    ]
    if with_count:
        out_type.append(jax.ShapeDtypeStruct((NC, HCR, 128), jnp.int32))
        scratch += [
            pltpu.VMEM((HCR, 128), jnp.int32),
            pltpu.VMEM((16, 16), jnp.int32),
            pltpu.VMEM((16, 128), jnp.int32),
            pltpu.VMEM((16, 128), jnp.int32),
            pltpu.VMEM_SHARED((NS, HCR, 128), jnp.int32),
        ]

    def body(hA, hB, src_hbm, dst_hbm, aggA, aggB, *rest):
        if with_count:
            (cnt_hbm, gbuf0, gbuf1, sidx0, sidx1, didx0, didx1, acc,
             gsem0, gsem1, hist, eye, macc, mtmp, hshared) = rest
        else:
            (gbuf0, gbuf1, sidx0, sidx1, didx0, didx1, acc,
             gsem0, gsem1) = rest
        c = lax.axis_index("c")
        s = lax.axis_index("s")
        zero16 = jnp.zeros((16,), jnp.float32)
        zero16i = jnp.zeros((16,), jnp.int32)
        iota16 = lax.iota(jnp.int32, 16)

        @pl.loop(0, K)
        def _(i):
            @pl.loop(0, DH // 16)
            def _(j):
                gbuf0[i, pl.ds(j * 16, 16)] = zero16

        @pl.loop(0, 8)
        def _(i):
            r0 = s * 640 + i * 80

            @pl.when(r0 < N)
            def _():
                pltpu.sync_copy(gbuf0, acc.at[pl.ds(r0, 80)])

        if with_count:
            @pl.loop(0, HCR)
            def _(i):
                @pl.loop(0, 8)
                def _(j):
                    hist[i, pl.ds(j * 16, 16)] = zero16i

            for i in range(16):
                eye[i, :] = jnp.maximum(1 - jnp.abs(iota16 - i), 0)

        plsc.subcore_barrier()

        def bump_chunk(didx):
            if not with_count:
                return

            @pl.loop(0, K // 16)
            def _(kk):
                dv = didx[pl.ds(kk * 16, 16)]
                for j in range(16):
                    d = dv[j]
                    row = lax.shift_right_logical(d, 7)
                    col = lax.bitwise_and(lax.shift_right_logical(d, 4),
                                          7) * 16
                    lane = lax.bitwise_and(d, 15)
                    sl = pl.ds(col, 16)
                    hist[row, sl] = hist[row, sl] + eye[lane, :]

        def run(h_hbm, agg_hbm):
            ebase = s * EDGES_PER_TILE

            def load_idx(i, sidx, didx):
                off = ebase + i * K
                pltpu.sync_copy(src_hbm.at[pl.ds(off, K)], sidx)
                pltpu.sync_copy(dst_hbm.at[pl.ds(off, K)], didx)

            load_idx(0, sidx0, didx0)
            pltpu.async_copy(h_hbm.at[sidx0], gbuf0, gsem0)

            @pl.loop(0, (NCHUNK - 1) // 2)
            def _(ih):
                i = ih * 2
                load_idx(i + 1, sidx1, didx1)
                pltpu.async_copy(h_hbm.at[sidx1], gbuf1, gsem1)
                pltpu.make_async_copy(h_hbm.at[sidx0], gbuf0, gsem0).wait()
                pltpu.sync_copy(gbuf0, acc.at[didx0], add=True)
                bump_chunk(didx0)
                load_idx(i + 2, sidx0, didx0)
                pltpu.async_copy(h_hbm.at[sidx0], gbuf0, gsem0)
                pltpu.make_async_copy(h_hbm.at[sidx1], gbuf1, gsem1).wait()
                pltpu.sync_copy(gbuf1, acc.at[didx1], add=True)
                bump_chunk(didx1)

            pltpu.make_async_copy(h_hbm.at[sidx0], gbuf0, gsem0).wait()
            pltpu.sync_copy(gbuf0, acc.at[didx0], add=True)
            bump_chunk(didx0)

            plsc.subcore_barrier()

            @pl.when(s < 2)
            def _():
                rows = pl.ds(s * (N // 2), N // 2)
                pltpu.sync_copy(acc.at[rows], agg_hbm.at[rows])

        @pl.when(c == 0)
        def _():
            run(hA, aggA)

        @pl.when(c == 1)
        def _():
            run(hB, aggB)

        if with_count:
            pltpu.sync_copy(hist, hshared.at[s])
            plsc.subcore_barrier()

            @pl.when(s < MT)
            def _():
                rows = pl.ds(s * 16, 16)

                @pl.loop(0, 16)
                def _(r):
                    @pl.loop(0, 8)
                    def _(j):
                        macc[r, pl.ds(j * 16, 16)] = zero16i

                @pl.loop(0, NS)
                def _(t):
                    pltpu.sync_copy(hshared.at[t, rows], mtmp)

                    @pl.loop(0, 16)
                    def _(r):
                        @pl.loop(0, 8)
                        def _(j):
                            sl = pl.ds(j * 16, 16)
                            macc[r, sl] = macc[r, sl] + mtmp[r, sl]

                pltpu.sync_copy(macc, cnt_hbm.at[c, rows])

    return functools.partial(
        pl.kernel, out_type=tuple(out_type), mesh=_mesh,
        scratch_types=scratch)(body)


_sc_segsum0 = _make_segsum(True)
_sc_segsum = _make_segsum(False)


BM = 1000


def _tc_sage_body(cnt0_ref, cnt1_ref, aggA_ref, aggB_ref, h_ref,
                  Wl_ref, Wr_ref, bl_ref, hn_ref, st_ref):
    i = pl.program_id(0)
    cnt = (cnt0_ref[...] + cnt1_ref[...]).astype(jnp.float32)
    inv = 1.0 / jnp.maximum(cnt, 1.0)
    agg = jnp.concatenate([aggA_ref[...], aggB_ref[...]], axis=1) * inv
    h = h_ref[...]
    out = jnp.dot(agg, Wl_ref[...], preferred_element_type=jnp.float32)
    out = out + jnp.dot(h, Wr_ref[...], preferred_element_type=jnp.float32)
    out = out + bl_ref[...]
    nrm = jnp.maximum(jnp.sqrt(jnp.sum(out * out, axis=1, keepdims=True)),
                      1e-12)
    y = out / nrm
    hn_ref[...] = y

    @pl.when(i == 0)
    def _():
        st_ref[...] = jnp.zeros_like(st_ref)

    s1 = jnp.sum(y, axis=0)[None, :]
    s2 = jnp.sum(y * y, axis=0)[None, :]
    st_ref[...] += jnp.concatenate(
        [s1, s2, jnp.zeros((6, D), jnp.float32)], axis=0)


_tc_sage = pl.pallas_call(
    _tc_sage_body,
    grid=(N // BM,),
    in_specs=[
        pl.BlockSpec((BM, 1), lambda i: (i, 0)),
        pl.BlockSpec((BM, 1), lambda i: (i, 0)),
        pl.BlockSpec((BM, DH), lambda i: (i, 0)),
        pl.BlockSpec((BM, DH), lambda i: (i, 0)),
        pl.BlockSpec((BM, D), lambda i: (i, 0)),
        pl.BlockSpec((D, D), lambda i: (0, 0)),
        pl.BlockSpec((D, D), lambda i: (0, 0)),
        pl.BlockSpec((1, D), lambda i: (0, 0)),
    ],
    out_specs=[
        pl.BlockSpec((BM, D), lambda i: (i, 0)),
        pl.BlockSpec((8, D), lambda i: (0, 0)),
    ],
    out_shape=[
        jax.ShapeDtypeStruct((N, D), jnp.float32),
        jax.ShapeDtypeStruct((8, D), jnp.float32),
    ],
)


def _tc_bn_body(hn_ref, st_ref, g_ref, b_ref, yf_ref, yA_ref, yB_ref):
    st = st_ref[...]
    m = st[0:1, :] * (1.0 / N)
    ex2 = st[1:2, :] * (1.0 / N)
    v = ex2 - m * m
    scale = g_ref[...] * jax.lax.rsqrt(v + 1e-5)
    y = (hn_ref[...] - m) * scale + b_ref[...]
    y = jnp.maximum(y, 0.0)
    yf_ref[...] = y
    yA_ref[...] = y[:, :DH]
    yB_ref[...] = y[:, DH:]


_tc_bn = pl.pallas_call(
    _tc_bn_body,
    grid=(N // BM,),
    in_specs=[
        pl.BlockSpec((BM, D), lambda i: (i, 0)),
        pl.BlockSpec((8, D), lambda i: (0, 0)),
        pl.BlockSpec((1, D), lambda i: (0, 0)),
        pl.BlockSpec((1, D), lambda i: (0, 0)),
    ],
    out_specs=[
        pl.BlockSpec((BM, D), lambda i: (i, 0)),
        pl.BlockSpec((BM, DH), lambda i: (i, 0)),
        pl.BlockSpec((BM, DH), lambda i: (i, 0)),
    ],
    out_shape=[
        jax.ShapeDtypeStruct((N, D), jnp.float32),
        jax.ShapeDtypeStruct((N, DH), jnp.float32),
        jax.ShapeDtypeStruct((N, DH), jnp.float32),
    ],
)


def kernel(x, edge_index, Wl0, bl0, Wr0, Wl1, bl1, Wr1, Wl2, bl2, Wr2,
           g0, b0, g1, b1):
    src = edge_index[0]
    dst = edge_index[1]

    hA = x[:, :DH]
    hB = x[:, DH:]
    h = x
    cnt1 = jnp.zeros((N, 1), jnp.int32)
    layers = (
        (Wl0, bl0, Wr0, g0, b0),
        (Wl1, bl1, Wr1, g1, b1),
        (Wl2, bl2, Wr2, None, None),
    )
    hn = None
    for li, (Wl, bl, Wr, g, b) in enumerate(layers):
        if li == 0:
            aggA, aggB, cnt2 = _sc_segsum0(hA, hB, src, dst)
            cnt0 = cnt2[0].reshape(HCR * 128)[:N].reshape(N, 1)
        else:
            aggA, aggB = _sc_segsum(hA, hB, src, dst)
        hn, st = _tc_sage(cnt0, cnt1, aggA, aggB, h, Wl, Wr,
                          bl.reshape(1, D))
        if g is not None:
            h, hA, hB = _tc_bn(hn, st, g.reshape(1, D), b.reshape(1, D))
    return hn


# speedup vs baseline: 1.0473x; 1.0473x over previous
import functools

import jax
import jax.numpy as jnp
from jax import lax
from jax.experimental import pallas as pl
from jax.experimental.pallas import tpu as pltpu
from jax.experimental.pallas import tpu_sc as plsc

N = 10000
E = 160000
D = 256
DH = 128
NC = 2
NS = 16
ROWS_PER_TILE = N // NS
EDGES_PER_TILE = E // NS
K = 80
NCHUNK = EDGES_PER_TILE // K
CNT_EPT = E // (NC * NS)
ZR = 125

_mesh = plsc.VectorSubcoreMesh(core_axis_name="c", subcore_axis_name="s")


HCR = 80
MT = 5
CNT_EPT = E // (NC * NS)
_CNT_FULL = CNT_EPT // 16 * 16
_CNT_TAIL = CNT_EPT - _CNT_FULL


@functools.partial(
    pl.kernel,
    out_type=jax.ShapeDtypeStruct((NC, HCR, 128), jnp.int32),
    mesh=_mesh,
    scratch_types=[
        pltpu.VMEM((HCR, 128), jnp.int32),
        pltpu.VMEM((CNT_EPT + 16,), jnp.int32),
        pltpu.VMEM((16, 16), jnp.int32),
        pltpu.VMEM((16, 128), jnp.int32),
        pltpu.VMEM((16, 128), jnp.int32),
        pltpu.VMEM_SHARED((NS, HCR, 128), jnp.int32),
    ],
)
def _sc_count(dst_hbm, out_hbm, hist, dbuf, eye, macc, mtmp, shared):
    c = lax.axis_index("c")
    s = lax.axis_index("s")
    zero16 = jnp.zeros((16,), jnp.int32)
    iota16 = lax.iota(jnp.int32, 16)

    @pl.loop(0, HCR)
    def _(i):
        @pl.loop(0, 8)
        def _(j):
            hist[i, pl.ds(j * 16, 16)] = zero16

    for i in range(16):
        eye[i, :] = jnp.maximum(1 - jnp.abs(iota16 - i), 0)

    base = (c * NS + s) * CNT_EPT
    pltpu.sync_copy(dst_hbm.at[pl.ds(base, CNT_EPT)],
                    dbuf.at[pl.ds(0, CNT_EPT)])

    def bump(d):
        row = lax.shift_right_logical(d, 7)
        col = lax.bitwise_and(lax.shift_right_logical(d, 4), 7) * 16
        lane = lax.bitwise_and(d, 15)
        sl = pl.ds(col, 16)
        hist[row, sl] = hist[row, sl] + eye[lane, :]

    @pl.loop(0, _CNT_FULL // 16)
    def _(k):
        dv = dbuf[pl.ds(k * 16, 16)]
        for j in range(16):
            bump(dv[j])

    dv_tail = dbuf[pl.ds(_CNT_FULL, 16)]
    for j in range(_CNT_TAIL):
        bump(dv_tail[j])

    pltpu.sync_copy(hist, shared.at[s])
    plsc.subcore_barrier()

    @pl.when(s < MT)
    def _():
        rows = pl.ds(s * 16, 16)

        @pl.loop(0, 16)
        def _(r):
            @pl.loop(0, 8)
            def _(j):
                macc[r, pl.ds(j * 16, 16)] = zero16

        @pl.loop(0, NS)
        def _(t):
            pltpu.sync_copy(shared.at[t, rows], mtmp)

            @pl.loop(0, 16)
            def _(r):
                @pl.loop(0, 8)
                def _(j):
                    sl = pl.ds(j * 16, 16)
                    macc[r, sl] = macc[r, sl] + mtmp[r, sl]

        pltpu.sync_copy(macc, out_hbm.at[c, rows])


@functools.partial(
    pl.kernel,
    out_type=(
        jax.ShapeDtypeStruct((N, DH), jnp.float32),
        jax.ShapeDtypeStruct((N, DH), jnp.float32),
    ),
    mesh=_mesh,
    scratch_types=[
        pltpu.VMEM((K, DH), jnp.float32),
        pltpu.VMEM((K, DH), jnp.float32),
        pltpu.VMEM((K,), jnp.int32),
        pltpu.VMEM((K,), jnp.int32),
        pltpu.VMEM((K,), jnp.int32),
        pltpu.VMEM((K,), jnp.int32),
        pltpu.VMEM((80, DH), jnp.float32),
        pltpu.VMEM_SHARED((N, DH), jnp.float32),
        pltpu.SemaphoreType.DMA,
        pltpu.SemaphoreType.DMA,
    ],
)
def _sc_segsum(hA, hB, src_hbm, dst_hbm, aggA, aggB,
               gbuf0, gbuf1, sidx0, sidx1, didx0, didx1, zbuf, acc,
               gsem0, gsem1):
    c = lax.axis_index("c")
    s = lax.axis_index("s")
    zero16 = jnp.zeros((16,), jnp.float32)

    @pl.loop(0, 80)
    def _(i):
        @pl.loop(0, DH // 16)
        def _(j):
            zbuf[i, pl.ds(j * 16, 16)] = zero16

    @pl.loop(0, 8)
    def _(i):
        r0 = s * 640 + i * 80

        @pl.when(r0 < N)
        def _():
            pltpu.sync_copy(zbuf, acc.at[pl.ds(r0, 80)])

    plsc.subcore_barrier()

    def run(h_hbm, agg_hbm):
        ebase = s * EDGES_PER_TILE

        def load_idx(i, sidx, didx):
            off = ebase + i * K
            pltpu.sync_copy(src_hbm.at[pl.ds(off, K)], sidx)
            pltpu.sync_copy(dst_hbm.at[pl.ds(off, K)], didx)

        load_idx(0, sidx0, didx0)
        pltpu.async_copy(h_hbm.at[sidx0], gbuf0, gsem0)

        @pl.loop(0, (NCHUNK - 1) // 2)
        def _(ih):
            i = ih * 2
            load_idx(i + 1, sidx1, didx1)
            pltpu.async_copy(h_hbm.at[sidx1], gbuf1, gsem1)
            pltpu.make_async_copy(h_hbm.at[sidx0], gbuf0, gsem0).wait()
            pltpu.sync_copy(gbuf0, acc.at[didx0], add=True)
            load_idx(i + 2, sidx0, didx0)
            pltpu.async_copy(h_hbm.at[sidx0], gbuf0, gsem0)
            pltpu.make_async_copy(h_hbm.at[sidx1], gbuf1, gsem1).wait()
            pltpu.sync_copy(gbuf1, acc.at[didx1], add=True)

        pltpu.make_async_copy(h_hbm.at[sidx0], gbuf0, gsem0).wait()
        pltpu.sync_copy(gbuf0, acc.at[didx0], add=True)

        plsc.subcore_barrier()

        @pl.when(s < 2)
        def _():
            rows = pl.ds(s * (N // 2), N // 2)
            pltpu.sync_copy(acc.at[rows], agg_hbm.at[rows])

    @pl.when(c == 0)
    def _():
        run(hA, aggA)

    @pl.when(c == 1)
    def _():
        run(hB, aggB)


BM = 1000


def _tc_sage_body(cnt0_ref, cnt1_ref, aggA_ref, aggB_ref, h_ref,
                  Wl_ref, Wr_ref, bl_ref, hn_ref, st_ref):
    i = pl.program_id(0)
    cnt = (cnt0_ref[...] + cnt1_ref[...]).astype(jnp.float32)
    inv = 1.0 / jnp.maximum(cnt, 1.0)
    agg = jnp.concatenate([aggA_ref[...], aggB_ref[...]], axis=1) * inv
    h = h_ref[...]
    out = jnp.dot(agg, Wl_ref[...], preferred_element_type=jnp.float32)
    out = out + jnp.dot(h, Wr_ref[...], preferred_element_type=jnp.float32)
    out = out + bl_ref[...]
    nrm = jnp.maximum(jnp.sqrt(jnp.sum(out * out, axis=1, keepdims=True)),
                      1e-12)
    y = out / nrm
    hn_ref[...] = y

    @pl.when(i == 0)
    def _():
        st_ref[...] = jnp.zeros_like(st_ref)

    s1 = jnp.sum(y, axis=0)[None, :]
    s2 = jnp.sum(y * y, axis=0)[None, :]
    st_ref[...] += jnp.concatenate(
        [s1, s2, jnp.zeros((6, D), jnp.float32)], axis=0)


_tc_sage = pl.pallas_call(
    _tc_sage_body,
    grid=(N // BM,),
    in_specs=[
        pl.BlockSpec((BM, 1), lambda i: (i, 0)),
        pl.BlockSpec((BM, 1), lambda i: (i, 0)),
        pl.BlockSpec((BM, DH), lambda i: (i, 0)),
        pl.BlockSpec((BM, DH), lambda i: (i, 0)),
        pl.BlockSpec((BM, D), lambda i: (i, 0)),
        pl.BlockSpec((D, D), lambda i: (0, 0)),
        pl.BlockSpec((D, D), lambda i: (0, 0)),
        pl.BlockSpec((1, D), lambda i: (0, 0)),
    ],
    out_specs=[
        pl.BlockSpec((BM, D), lambda i: (i, 0)),
        pl.BlockSpec((8, D), lambda i: (0, 0)),
    ],
    out_shape=[
        jax.ShapeDtypeStruct((N, D), jnp.float32),
        jax.ShapeDtypeStruct((8, D), jnp.float32),
    ],
)


def _tc_bn_body(hn_ref, st_ref, g_ref, b_ref, yf_ref, yA_ref, yB_ref):
    st = st_ref[...]
    m = st[0:1, :] * (1.0 / N)
    ex2 = st[1:2, :] * (1.0 / N)
    v = ex2 - m * m
    scale = g_ref[...] * jax.lax.rsqrt(v + 1e-5)
    y = (hn_ref[...] - m) * scale + b_ref[...]
    y = jnp.maximum(y, 0.0)
    yf_ref[...] = y
    yA_ref[...] = y[:, :DH]
    yB_ref[...] = y[:, DH:]


_tc_bn = pl.pallas_call(
    _tc_bn_body,
    grid=(N // BM,),
    in_specs=[
        pl.BlockSpec((BM, D), lambda i: (i, 0)),
        pl.BlockSpec((8, D), lambda i: (0, 0)),
        pl.BlockSpec((1, D), lambda i: (0, 0)),
        pl.BlockSpec((1, D), lambda i: (0, 0)),
    ],
    out_specs=[
        pl.BlockSpec((BM, D), lambda i: (i, 0)),
        pl.BlockSpec((BM, DH), lambda i: (i, 0)),
        pl.BlockSpec((BM, DH), lambda i: (i, 0)),
    ],
    out_shape=[
        jax.ShapeDtypeStruct((N, D), jnp.float32),
        jax.ShapeDtypeStruct((N, DH), jnp.float32),
        jax.ShapeDtypeStruct((N, DH), jnp.float32),
    ],
)


def kernel(x, edge_index, Wl0, bl0, Wr0, Wl1, bl1, Wr1, Wl2, bl2, Wr2,
           g0, b0, g1, b1):
    src = edge_index[0]
    dst = edge_index[1]

    cnt2 = _sc_count(dst)
    cnt0 = cnt2[0].reshape(HCR * 128)[:N].reshape(N, 1)
    cnt1 = cnt2[1].reshape(HCR * 128)[:N].reshape(N, 1)

    hA = x[:, :DH]
    hB = x[:, DH:]
    h = x
    layers = (
        (Wl0, bl0, Wr0, g0, b0),
        (Wl1, bl1, Wr1, g1, b1),
        (Wl2, bl2, Wr2, None, None),
    )
    hn = None
    for Wl, bl, Wr, g, b in layers:
        aggA, aggB = _sc_segsum(hA, hB, src, dst)
        hn, st = _tc_sage(cnt0, cnt1, aggA, aggB, h, Wl, Wr,
                          bl.reshape(1, D))
        if g is not None:
            h, hA, hB = _tc_bn(hn, st, g.reshape(1, D), b.reshape(1, D))
    return hn


# speedup vs baseline: 1.0541x; 1.0066x over previous
import functools

import jax
import jax.numpy as jnp
from jax import lax
from jax.experimental import pallas as pl
from jax.experimental.pallas import tpu as pltpu
from jax.experimental.pallas import tpu_sc as plsc

N = 10000
E = 160000
D = 256
DH = 128
NC = 2
NS = 16
ROWS_PER_TILE = N // NS
EDGES_PER_TILE = E // NS
K = 80
NCHUNK = EDGES_PER_TILE // K
CNT_EPT = E // (NC * NS)
ZR = 125

_mesh = plsc.VectorSubcoreMesh(core_axis_name="c", subcore_axis_name="s")


HCR = 80
MT = 5
CNT_EPT = E // (NC * NS)
_CNT_FULL = CNT_EPT // 16 * 16
_CNT_TAIL = CNT_EPT - _CNT_FULL


@functools.partial(
    pl.kernel,
    out_type=jax.ShapeDtypeStruct((NC, HCR, 128), jnp.int32),
    mesh=_mesh,
    scratch_types=[
        pltpu.VMEM((HCR, 128), jnp.int32),
        pltpu.VMEM((CNT_EPT + 16,), jnp.int32),
        pltpu.VMEM((16, 16), jnp.int32),
        pltpu.VMEM((16, 128), jnp.int32),
        pltpu.VMEM((16, 128), jnp.int32),
        pltpu.VMEM_SHARED((NS, HCR, 128), jnp.int32),
    ],
)
def _sc_count(dst_hbm, out_hbm, hist, dbuf, eye, macc, mtmp, shared):
    c = lax.axis_index("c")
    s = lax.axis_index("s")
    zero16 = jnp.zeros((16,), jnp.int32)
    iota16 = lax.iota(jnp.int32, 16)

    @pl.loop(0, HCR)
    def _(i):
        @pl.loop(0, 8)
        def _(j):
            hist[i, pl.ds(j * 16, 16)] = zero16

    for i in range(16):
        eye[i, :] = jnp.maximum(1 - jnp.abs(iota16 - i), 0)

    base = (c * NS + s) * CNT_EPT
    pltpu.sync_copy(dst_hbm.at[pl.ds(base, CNT_EPT)],
                    dbuf.at[pl.ds(0, CNT_EPT)])

    def bump(d):
        row = lax.shift_right_logical(d, 7)
        col = lax.bitwise_and(lax.shift_right_logical(d, 4), 7) * 16
        lane = lax.bitwise_and(d, 15)
        sl = pl.ds(col, 16)
        hist[row, sl] = hist[row, sl] + eye[lane, :]

    @pl.loop(0, _CNT_FULL // 16)
    def _(k):
        dv = dbuf[pl.ds(k * 16, 16)]
        for j in range(16):
            bump(dv[j])

    dv_tail = dbuf[pl.ds(_CNT_FULL, 16)]
    for j in range(_CNT_TAIL):
        bump(dv_tail[j])

    pltpu.sync_copy(hist, shared.at[s])
    plsc.subcore_barrier()

    @pl.when(s < MT)
    def _():
        rows = pl.ds(s * 16, 16)

        @pl.loop(0, 16)
        def _(r):
            @pl.loop(0, 8)
            def _(j):
                macc[r, pl.ds(j * 16, 16)] = zero16

        @pl.loop(0, NS)
        def _(t):
            pltpu.sync_copy(shared.at[t, rows], mtmp)

            @pl.loop(0, 16)
            def _(r):
                @pl.loop(0, 8)
                def _(j):
                    sl = pl.ds(j * 16, 16)
                    macc[r, sl] = macc[r, sl] + mtmp[r, sl]

        pltpu.sync_copy(macc, out_hbm.at[c, rows])


@functools.partial(
    pl.kernel,
    out_type=(
        jax.ShapeDtypeStruct((N, DH), jnp.float32),
        jax.ShapeDtypeStruct((N, DH), jnp.float32),
    ),
    mesh=_mesh,
    scratch_types=[
        pltpu.VMEM((K, DH), jnp.float32),
        pltpu.VMEM((K, DH), jnp.float32),
        pltpu.VMEM((K,), jnp.int32),
        pltpu.VMEM((K,), jnp.int32),
        pltpu.VMEM((K,), jnp.int32),
        pltpu.VMEM((K,), jnp.int32),
        pltpu.VMEM((80, DH), jnp.float32),
        pltpu.VMEM_SHARED((N, DH), jnp.float32),
        pltpu.SemaphoreType.DMA,
        pltpu.SemaphoreType.DMA,
    ],
)
def _sc_segsum(hA, hB, src_hbm, dst_hbm, aggA, aggB,
               gbuf0, gbuf1, sidx0, sidx1, didx0, didx1, zbuf, acc,
               gsem0, gsem1):
    c = lax.axis_index("c")
    s = lax.axis_index("s")
    zero16 = jnp.zeros((16,), jnp.float32)

    @pl.loop(0, 80)
    def _(i):
        @pl.loop(0, DH // 16)
        def _(j):
            zbuf[i, pl.ds(j * 16, 16)] = zero16

    @pl.loop(0, 8)
    def _(i):
        r0 = s * 640 + i * 80

        @pl.when(r0 < N)
        def _():
            pltpu.sync_copy(zbuf, acc.at[pl.ds(r0, 80)])

    plsc.subcore_barrier()

    def run(h_hbm, agg_hbm):
        ebase = s * EDGES_PER_TILE

        def load_idx(i, sidx, didx):
            off = ebase + i * K
            pltpu.sync_copy(src_hbm.at[pl.ds(off, K)], sidx)
            pltpu.sync_copy(dst_hbm.at[pl.ds(off, K)], didx)

        load_idx(0, sidx0, didx0)
        pltpu.async_copy(h_hbm.at[sidx0], gbuf0, gsem0)

        @pl.loop(0, (NCHUNK - 1) // 2)
        def _(ih):
            i = ih * 2
            load_idx(i + 1, sidx1, didx1)
            pltpu.async_copy(h_hbm.at[sidx1], gbuf1, gsem1)
            pltpu.make_async_copy(h_hbm.at[sidx0], gbuf0, gsem0).wait()
            pltpu.sync_copy(gbuf0, acc.at[didx0], add=True)
            load_idx(i + 2, sidx0, didx0)
            pltpu.async_copy(h_hbm.at[sidx0], gbuf0, gsem0)
            pltpu.make_async_copy(h_hbm.at[sidx1], gbuf1, gsem1).wait()
            pltpu.sync_copy(gbuf1, acc.at[didx1], add=True)

        pltpu.make_async_copy(h_hbm.at[sidx0], gbuf0, gsem0).wait()
        pltpu.sync_copy(gbuf0, acc.at[didx0], add=True)

        plsc.subcore_barrier()

        @pl.when(s < 2)
        def _():
            rows = pl.ds(s * (N // 2), N // 2)
            pltpu.sync_copy(acc.at[rows], agg_hbm.at[rows])

    @pl.when(c == 0)
    def _():
        run(hA, aggA)

    @pl.when(c == 1)
    def _():
        run(hB, aggB)


BM = 1000


def _tc_sage_body(cnt0_ref, cnt1_ref, aggA_ref, aggB_ref, hA_ref, hB_ref,
                  Wl_ref, Wr_ref, bl_ref, hn_ref, st_ref):
    i = pl.program_id(0)
    cnt = (cnt0_ref[...] + cnt1_ref[...]).astype(jnp.float32)
    inv = 1.0 / jnp.maximum(cnt, 1.0)
    agg = jnp.concatenate([aggA_ref[...], aggB_ref[...]], axis=1) * inv
    h = jnp.concatenate([hA_ref[...], hB_ref[...]], axis=1)
    out = jnp.dot(agg, Wl_ref[...], preferred_element_type=jnp.float32)
    out = out + jnp.dot(h, Wr_ref[...], preferred_element_type=jnp.float32)
    out = out + bl_ref[...]
    nrm = jnp.maximum(jnp.sqrt(jnp.sum(out * out, axis=1, keepdims=True)),
                      1e-12)
    y = out / nrm
    hn_ref[...] = y

    @pl.when(i == 0)
    def _():
        st_ref[...] = jnp.zeros_like(st_ref)

    s1 = jnp.sum(y, axis=0)[None, :]
    s2 = jnp.sum(y * y, axis=0)[None, :]
    st_ref[...] += jnp.concatenate(
        [s1, s2, jnp.zeros((6, D), jnp.float32)], axis=0)


_tc_sage = pl.pallas_call(
    _tc_sage_body,
    grid=(N // BM,),
    in_specs=[
        pl.BlockSpec((BM, 1), lambda i: (i, 0)),
        pl.BlockSpec((BM, 1), lambda i: (i, 0)),
        pl.BlockSpec((BM, DH), lambda i: (i, 0)),
        pl.BlockSpec((BM, DH), lambda i: (i, 0)),
        pl.BlockSpec((BM, DH), lambda i: (i, 0)),
        pl.BlockSpec((BM, DH), lambda i: (i, 0)),
        pl.BlockSpec((D, D), lambda i: (0, 0)),
        pl.BlockSpec((D, D), lambda i: (0, 0)),
        pl.BlockSpec((1, D), lambda i: (0, 0)),
    ],
    out_specs=[
        pl.BlockSpec((BM, D), lambda i: (i, 0)),
        pl.BlockSpec((8, D), lambda i: (0, 0)),
    ],
    out_shape=[
        jax.ShapeDtypeStruct((N, D), jnp.float32),
        jax.ShapeDtypeStruct((8, D), jnp.float32),
    ],
)


def _tc_bn_body(hn_ref, st_ref, g_ref, b_ref, yA_ref, yB_ref):
    st = st_ref[...]
    m = st[0:1, :] * (1.0 / N)
    ex2 = st[1:2, :] * (1.0 / N)
    v = ex2 - m * m
    scale = g_ref[...] * jax.lax.rsqrt(v + 1e-5)
    y = (hn_ref[...] - m) * scale + b_ref[...]
    y = jnp.maximum(y, 0.0)
    yA_ref[...] = y[:, :DH]
    yB_ref[...] = y[:, DH:]


_tc_bn = pl.pallas_call(
    _tc_bn_body,
    grid=(N // BM,),
    in_specs=[
        pl.BlockSpec((BM, D), lambda i: (i, 0)),
        pl.BlockSpec((8, D), lambda i: (0, 0)),
        pl.BlockSpec((1, D), lambda i: (0, 0)),
        pl.BlockSpec((1, D), lambda i: (0, 0)),
    ],
    out_specs=[
        pl.BlockSpec((BM, DH), lambda i: (i, 0)),
        pl.BlockSpec((BM, DH), lambda i: (i, 0)),
    ],
    out_shape=[
        jax.ShapeDtypeStruct((N, DH), jnp.float32),
        jax.ShapeDtypeStruct((N, DH), jnp.float32),
    ],
)


def kernel(x, edge_index, Wl0, bl0, Wr0, Wl1, bl1, Wr1, Wl2, bl2, Wr2,
           g0, b0, g1, b1):
    src = edge_index[0]
    dst = edge_index[1]

    cnt2 = _sc_count(dst)
    cnt0 = cnt2[0].reshape(HCR * 128)[:N].reshape(N, 1)
    cnt1 = cnt2[1].reshape(HCR * 128)[:N].reshape(N, 1)

    hA = x[:, :DH]
    hB = x[:, DH:]
    layers = (
        (Wl0, bl0, Wr0, g0, b0),
        (Wl1, bl1, Wr1, g1, b1),
        (Wl2, bl2, Wr2, None, None),
    )
    hn = None
    for Wl, bl, Wr, g, b in layers:
        aggA, aggB = _sc_segsum(hA, hB, src, dst)
        hn, st = _tc_sage(cnt0, cnt1, aggA, aggB, hA, hB, Wl, Wr,
                          bl.reshape(1, D))
        if g is not None:
            hA, hB = _tc_bn(hn, st, g.reshape(1, D), b.reshape(1, D))
    return hn
